# baseline (speedup 1.0000x reference)
import jax
import jax.numpy as jnp
from jax import lax
from jax.experimental import pallas as pl
from jax.experimental.pallas import tpu as pltpu
from jax.experimental.pallas import tpu_sc as plsc

ALPHA = 0.4
BETA = 0.3
C0 = 1.0 - ALPHA - BETA
V = 1000
S = 2048
B = 4
T_HASH = 8192
N = S * B
NW = 32
PER_W = N // NW
CB = 16
NCH = PER_W // CB
ROWP = 1024
NSL = 63
EP_R = 1024


def _sc_body(text_h, uni_h, big_h, tri_h, out_h,
             txt_v, curi_v, trii_v, unis_v, big_v, tri_v, out_v,
             sem_g0, sem_g1, sem_o0, sem_o1):
    sem_g = (sem_g0, sem_g1)
    sem_o = (sem_o0, sem_o1)
    cid = lax.axis_index("c")
    sid = lax.axis_index("s")
    wid = sid * 2 + cid
    base = wid * PER_W

    pltpu.sync_copy(text_h, txt_v)
    pltpu.sync_copy(uni_h, unis_v.at[pl.ds(0, V)])

    @plsc.parallel_loop(0, NSL, unroll=4)
    def scale_uni(j):
        off = j * 16
        unis_v[pl.ds(off, 16)] = C0 * unis_v[pl.ds(off, 16)]

    def idx_body(s_, _):
        cur = txt_v[pl.ds(base + s_ * 16 + 8, 16)]
        prev = txt_v[pl.ds(base + s_ * 16 + 4, 16)]
        tri = (prev * V + cur) & (T_HASH - 1)
        curi_v[s_, :] = cur
        trii_v[s_, :] = tri
        return 0
    lax.fori_loop(0, PER_W // 16, idx_body, 0)

    def gathers(c, buf):
        cb = pltpu.make_async_copy(big_h.at[curi_v.at[c]], big_v.at[buf],
                                   sem_g[buf])
        ct = pltpu.make_async_copy(tri_h.at[trii_v.at[c]], tri_v.at[buf],
                                   sem_g[buf])
        return cb, ct

    def out_copy(c, buf):
        return pltpu.make_async_copy(out_v.at[buf],
                                     out_h.at[pl.ds(base + c * CB, CB)],
                                     sem_o[buf])

    def chunk_body(c, buf):
        bv = big_v.at[buf]
        tv = tri_v.at[buf]
        ov = out_v.at[buf]

        def row_body(r, _):
            k = base + c * CB + r
            betak = jnp.where(jnp.broadcast_to(k, (16,)) >= 2 * B,
                              jnp.float32(BETA), jnp.float32(0.0))

            @plsc.parallel_loop(0, 62, unroll=4)
            def p1(j):
                ct_ = j // 8
                cl = (j % 8) * 16
                off = j * 16
                p = (unis_v[pl.ds(off, 16)]
                     + ALPHA * bv[r, ct_, pl.ds(cl, 16)]
                     + betak * tv[r, pl.ds(off, 16)])
                ov[r, ct_, pl.ds(cl, 16)] = p
            p = (unis_v[pl.ds(984, 16)]
                 + ALPHA * bv[r, 7, pl.ds(88, 16)]
                 + betak * tv[r, pl.ds(984, 16)])
            ov[r, 7, pl.ds(88, 16)] = p
            return 0
        lax.fori_loop(0, CB, row_body, 0)

    g0b, g0t = gathers(0, 0)
    g0b.start()
    g0t.start()
    for c in range(NCH):
        buf = c % 2
        if c + 1 < NCH:
            nb, nt = gathers(c + 1, 1 - buf)
            nb.start()
            nt.start()
        gb, gt = gathers(c, buf)
        gb.wait()
        gt.wait()
        if c >= 2:
            out_copy(c - 2, buf).wait()
        chunk_body(c, buf)
        out_copy(c, buf).start()
    out_copy(NCH - 2, NCH % 2).wait()
    out_copy(NCH - 1, (NCH - 1) % 2).wait()


def _tc_epilogue(p_ref, o_ref):
    x = p_ref[...]
    ct_ = lax.broadcasted_iota(jnp.int32, x.shape, 1)
    cl = lax.broadcasted_iota(jnp.int32, x.shape, 2)
    valid = (ct_ * 128 + cl) < V
    xz = jnp.where(valid, x, 0.0)
    s = jnp.sum(xz, axis=(1, 2), keepdims=True) + 1e-10
    q = jnp.log(1e-10 + xz / s)
    y = q.reshape(EP_R, ROWP)[:, :V].reshape(EP_R // B, B, V)
    o_ref[...] = y


@jax.jit
def kernel(text, unigram, bigram_table, trigram_table):
    textf = jnp.pad(text.reshape(N), (8, 0))
    big3 = jnp.pad(bigram_table, ((0, 0), (0, 24))).reshape(V, 8, 128)

    mesh = plsc.VectorSubcoreMesh(core_axis_name="c", subcore_axis_name="s")
    p3 = pl.kernel(
        _sc_body,
        out_type=jax.ShapeDtypeStruct((N, 8, 128), jnp.float32),
        mesh=mesh,
        compiler_params=pltpu.CompilerParams(
            needs_layout_passes=False, use_tc_tiling_on_sc=False),
        scratch_types=[
            pltpu.VMEM((N + 8,), jnp.int32),
            pltpu.VMEM((NCH, CB), jnp.int32),
            pltpu.VMEM((NCH, CB), jnp.int32),
            pltpu.VMEM((ROWP,), jnp.float32),
            pltpu.VMEM((2, CB, 8, 128), jnp.float32),
            pltpu.VMEM((2, CB, V), jnp.float32),
            pltpu.VMEM((2, CB, 8, 128), jnp.float32),
            pltpu.SemaphoreType.DMA,
            pltpu.SemaphoreType.DMA,
            pltpu.SemaphoreType.DMA,
            pltpu.SemaphoreType.DMA,
        ],
    )(textf, unigram, big3, trigram_table)

    out = pl.pallas_call(
        _tc_epilogue,
        grid=(N // EP_R,),
        in_specs=[pl.BlockSpec((EP_R, 8, 128), lambda i: (i, 0, 0))],
        out_specs=pl.BlockSpec((EP_R // B, B, V), lambda i: (i, 0, 0)),
        out_shape=jax.ShapeDtypeStruct((S, B, V), jnp.float32),
        compiler_params=pltpu.CompilerParams(
            dimension_semantics=("arbitrary",)),
    )(p3)
    return out


# speedup vs baseline: 1.0887x; 1.0887x over previous
import jax
import jax.numpy as jnp
from jax import lax
from jax.experimental import pallas as pl
from jax.experimental.pallas import tpu as pltpu
from jax.experimental.pallas import tpu_sc as plsc

ALPHA = 0.4
BETA = 0.3
C0 = 1.0 - ALPHA - BETA
V = 1000
S = 2048
B = 4
T_HASH = 8192
N = S * B
NW = 32
PER_W = N // NW
CB = 16
NCH = PER_W // CB
ROWP = 1024
NSL = 63
EP_R = 1024


def _sc_body(text_h, uni_h, big_h, tri_h, out_h,
             txt_v, curi_v, trii_v, unis_v, big_v, tri_v, out_v,
             sem_g0, sem_g1, sem_o0, sem_o1):
    sem_g = (sem_g0, sem_g1)
    sem_o = (sem_o0, sem_o1)
    cid = lax.axis_index("c")
    sid = lax.axis_index("s")
    wid = sid * 2 + cid
    base = wid * PER_W

    pltpu.sync_copy(text_h, txt_v)
    pltpu.sync_copy(uni_h, unis_v.at[pl.ds(0, V)])

    @plsc.parallel_loop(0, NSL, unroll=4)
    def scale_uni(j):
        off = j * 16
        unis_v[pl.ds(off, 16)] = C0 * unis_v[pl.ds(off, 16)]

    def idx_body(s_, _):
        cur = txt_v[pl.ds(base + s_ * 16 + 8, 16)]
        prev = txt_v[pl.ds(base + s_ * 16 + 4, 16)]
        tri = (prev * V + cur) & (T_HASH - 1)
        curi_v[s_, :] = cur
        trii_v[s_, :] = tri
        return 0
    lax.fori_loop(0, PER_W // 16, idx_body, 0)

    def gathers(c, buf):
        cb = pltpu.make_async_copy(big_h.at[curi_v.at[c]], big_v.at[buf],
                                   sem_g[buf])
        ct = pltpu.make_async_copy(tri_h.at[trii_v.at[c]], tri_v.at[buf],
                                   sem_g[buf])
        return cb, ct

    def out_copy(c, buf):
        return pltpu.make_async_copy(out_v.at[buf],
                                     out_h.at[pl.ds(base + c * CB, CB)],
                                     sem_o[buf])

    def chunk_body(c, buf):
        bv = big_v.at[buf]
        tv = tri_v.at[buf]
        ov = out_v.at[buf]

        def row_body(r, _):
            k = base + c * CB + r
            betak = jnp.where(jnp.broadcast_to(k, (16,)) >= 2 * B,
                              jnp.float32(BETA), jnp.float32(0.0))

            @plsc.parallel_loop(0, NSL, unroll=4)
            def p1(j):
                ct_ = j // 8
                cl = (j % 8) * 16
                off = j * 16
                p = (unis_v[pl.ds(off, 16)]
                     + ALPHA * bv[r, ct_, pl.ds(cl, 16)]
                     + betak * tv[r, ct_, pl.ds(cl, 16)])
                ov[r, ct_, pl.ds(cl, 16)] = p
            return 0
        lax.fori_loop(0, CB, row_body, 0)

    g0b, g0t = gathers(0, 0)
    g0b.start()
    g0t.start()
    for c in range(NCH):
        buf = c % 2
        if c + 1 < NCH:
            nb, nt = gathers(c + 1, 1 - buf)
            nb.start()
            nt.start()
        gb, gt = gathers(c, buf)
        gb.wait()
        gt.wait()
        if c >= 2:
            out_copy(c - 2, buf).wait()
        chunk_body(c, buf)
        out_copy(c, buf).start()
    out_copy(NCH - 2, NCH % 2).wait()
    out_copy(NCH - 1, (NCH - 1) % 2).wait()


def _tc_epilogue(p_ref, o_ref):
    x = p_ref[...]
    ct_ = lax.broadcasted_iota(jnp.int32, x.shape, 1)
    cl = lax.broadcasted_iota(jnp.int32, x.shape, 2)
    valid = (ct_ * 128 + cl) < V
    xz = jnp.where(valid, x, 0.0)
    s = jnp.sum(xz, axis=(1, 2), keepdims=True) + 1e-10
    q = jnp.log(1e-10 + xz / s)
    y = q.reshape(EP_R, ROWP)[:, :V].reshape(EP_R // B, B, V)
    o_ref[...] = y


@jax.jit
def kernel(text, unigram, bigram_table, trigram_table):
    textf = jnp.pad(text.reshape(N), (8, 0))
    big3 = jnp.pad(bigram_table, ((0, 0), (0, 24))).reshape(V, 8, 128)
    tri3 = jnp.pad(trigram_table, ((0, 0), (0, 24))).reshape(T_HASH, 8, 128)

    mesh = plsc.VectorSubcoreMesh(core_axis_name="c", subcore_axis_name="s")
    p3 = pl.kernel(
        _sc_body,
        out_type=jax.ShapeDtypeStruct((N, 8, 128), jnp.float32),
        mesh=mesh,
        compiler_params=pltpu.CompilerParams(
            needs_layout_passes=False, use_tc_tiling_on_sc=False),
        scratch_types=[
            pltpu.VMEM((N + 8,), jnp.int32),
            pltpu.VMEM((NCH, CB), jnp.int32),
            pltpu.VMEM((NCH, CB), jnp.int32),
            pltpu.VMEM((ROWP,), jnp.float32),
            pltpu.VMEM((2, CB, 8, 128), jnp.float32),
            pltpu.VMEM((2, CB, 8, 128), jnp.float32),
            pltpu.VMEM((2, CB, 8, 128), jnp.float32),
            pltpu.SemaphoreType.DMA,
            pltpu.SemaphoreType.DMA,
            pltpu.SemaphoreType.DMA,
            pltpu.SemaphoreType.DMA,
        ],
    )(textf, unigram, big3, tri3)

    out = pl.pallas_call(
        _tc_epilogue,
        grid=(N // EP_R,),
        in_specs=[pl.BlockSpec((EP_R, 8, 128), lambda i: (i, 0, 0))],
        out_specs=pl.BlockSpec((EP_R // B, B, V), lambda i: (i, 0, 0)),
        out_shape=jax.ShapeDtypeStruct((S, B, V), jnp.float32),
        compiler_params=pltpu.CompilerParams(
            dimension_semantics=("arbitrary",)),
    )(p3)
    return out
